# baseline (speedup 1.0000x reference)
import functools

import jax
import jax.numpy as jnp
from jax import lax
from jax.experimental import pallas as pl
from jax.experimental.pallas import tpu as pltpu
from jax.experimental.pallas import tpu_sc as plsc

H, W, B, NB = 496, 432, 4, 50
HP = 512
INV = 1.25
RW = 40
NBOX = 224
NWORK = 32
BPW = NBOX // NWORK
NCH = RW * 4


def _boxparams_body(bx_ref, out_ref):
    cx = bx_ref[0, :]
    cy = bx_ref[1, :]
    cz = bx_ref[2, :]
    dx = bx_ref[3, :]
    dy = bx_ref[4, :]
    dz = bx_ref[5, :]
    rz = bx_ref[6, :]
    bidx = bx_ref[7, :]
    c = jnp.cos(rz)
    s = jnp.sin(rz)
    zok = jnp.logical_and(jnp.abs(cz) <= dz * 0.5, bidx >= 0.0)
    cxg = cx * INV
    cyg = cy * INV
    hxg = jnp.where(zok, dx * (0.5 * INV), -1.0)
    hyg = dy * (0.5 * INV)
    r0 = jnp.clip(jnp.floor(cyg) - 20.0, 0.0, float(W - RW))
    c0 = jnp.clip(jnp.floor((cxg - 18.0) / 16.0), 0.0, float((H - 64) // 16)) * 16.0
    bb = jnp.maximum(bidx, 0.0)
    out_ref[0, :] = cxg
    out_ref[1, :] = cyg
    out_ref[2, :] = c
    out_ref[3, :] = s
    out_ref[4, :] = hxg
    out_ref[5, :] = hyg
    out_ref[6, :] = r0
    out_ref[7, :] = c0
    out_ref[8, :] = bb * float(W) + r0


def _boxparams(boxes):
    bx = jnp.transpose(boxes, (2, 0, 1)).reshape(7, B * NB)
    bxp = jnp.zeros((8, NBOX), jnp.float32)
    bxp = bxp.at[:7, : B * NB].set(bx)
    bidx = jnp.where(
        jnp.arange(NBOX) < B * NB, jnp.arange(NBOX) // NB, -1
    ).astype(jnp.float32)
    bxp = bxp.at[7, :].set(bidx)
    return pl.pallas_call(
        _boxparams_body,
        out_shape=jax.ShapeDtypeStruct((9, NBOX), jnp.float32),
    )(bxp)


TW1 = 216


def _code_body(added_ref, orig_ref, code_ref):
    pred = added_ref[0, 0] + added_ref[0, 1] + added_ref[0, 2] + added_ref[0, 3]
    og = orig_ref[0, 1] + orig_ref[0, 2] + orig_ref[0, 3] + orig_ref[0, 4]
    code = jnp.where(pred != 0.0, 1, 0) + jnp.where(og != 0.0, 2, 0)
    code_ref[0] = jnp.pad(code, ((0, 0), (0, HP - H))).astype(jnp.int8)


def _code(added_t, orig_t):
    return pl.pallas_call(
        _code_body,
        grid=(B, W // TW1),
        in_specs=[
            pl.BlockSpec((1, 4, TW1, H), lambda b, w: (b, 0, w, 0)),
            pl.BlockSpec((1, 5, TW1, H), lambda b, w: (b, 0, w, 0)),
        ],
        out_specs=pl.BlockSpec((1, TW1, HP), lambda b, w: (b, w, 0)),
        out_shape=jax.ShapeDtypeStruct((B, W, HP), jnp.int8),
    )(added_t, orig_t)


def _splat_i(val):
    return jnp.full((16,), val, jnp.int32)


def _splat_f(val):
    return jnp.full((16,), val, jnp.float32)


def _sc_body(code_hbm, params_hbm, out_hbm, params_v, idx_v, dst, outbuf, sem0):
    wid = lax.axis_index("s") * 2 + lax.axis_index("c")
    pltpu.sync_copy(params_hbm, params_v)
    iota = lax.broadcasted_iota(jnp.int32, (16,), 0)
    iota_f = iota.astype(jnp.float32)
    widx = lax.shift_right_logical(iota, 2)
    shv = lax.bitwise_and(iota, 3) * 8
    total_v = jnp.zeros((16,), jnp.float32)

    for k in range(BPW):
        box = wid + NWORK * k
        bsp = _splat_i(box)

        def _p(row):
            return plsc.load_gather(params_v, [_splat_i(row), bsp])

        cxg = _p(0)
        cyg = _p(1)
        c = _p(2)
        s = _p(3)
        hxg = _p(4)
        hyg = _p(5)
        r0f = _p(6)
        c0f = _p(7)
        base_i = _p(8).astype(jnp.int32)
        w0 = lax.shift_right_logical(c0f.astype(jnp.int32), 2)

        idx_v[pl.ds(0, 16)] = base_i + iota
        idx_v[pl.ds(16, 16)] = base_i + iota + 16
        idx_v[pl.ds(24, 16)] = base_i + iota + 24

        pltpu.async_copy(code_hbm.at[idx_v], dst, sem0).wait()

        u0 = c0f + iota_f - cxg
        v0 = r0f - cyg

        def _chunk_step(it, accs):
            acc_u, acc_i = accs
            dv0 = _splat_i(2 * it).astype(jnp.float32)
            row0 = _splat_i(2 * it)
            for t in range(8):
                xt = t % 4
                v = v0 + (dv0 + float(t // 4))
                u = u0 + float(16 * xt)
                lx = u * c + v * s
                ly = v * c - u * s
                m = jnp.logical_and(jnp.abs(lx) <= hxg, jnp.abs(ly) <= hyg)
                wordv = plsc.load_gather(
                    dst, [row0 + (t // 4), w0 + (4 * xt) + widx])
                codev = lax.bitwise_and(
                    lax.shift_right_logical(wordv, shv), 255)
                acc_u = acc_u + jnp.where(
                    jnp.logical_and(m, codev != 0), 1.0, 0.0)
                acc_i = acc_i + jnp.where(
                    jnp.logical_and(m, codev == 3), 1.0, 0.0)
            return acc_u, acc_i

        acc_u, acc_i = lax.fori_loop(
            0, NCH // 8, _chunk_step,
            (jnp.zeros((16,), jnp.float32), jnp.zeros((16,), jnp.float32)))

        us = jnp.sum(acc_u)
        isum = jnp.sum(acc_i)
        total_v = total_v + _splat_f(isum) / jnp.maximum(_splat_f(us), 1.0)

    outbuf[...] = total_v
    pltpu.sync_copy(outbuf, out_hbm.at[wid])


def _sc_counts(codep, params):
    mesh = plsc.VectorSubcoreMesh(core_axis_name="c", subcore_axis_name="s")
    f = functools.partial(
        pl.kernel,
        mesh=mesh,
        compiler_params=pltpu.CompilerParams(
            needs_layout_passes=False, use_tc_tiling_on_sc=False),
        out_type=jax.ShapeDtypeStruct((NWORK, 16), jnp.float32),
        scratch_types=[
            pltpu.VMEM((9, NBOX), jnp.float32),
            pltpu.VMEM((RW,), jnp.int32),
            pltpu.VMEM((RW, HP // 4), jnp.int32),
            pltpu.VMEM((16,), jnp.float32),
            pltpu.SemaphoreType.DMA,
        ],
    )(_sc_body)
    return f(codep, params)


def kernel(added_points, original_points, boxes):
    params = _boxparams(boxes)
    added_t = jnp.swapaxes(added_points, 2, 3)
    orig_t = jnp.swapaxes(original_points, 2, 3)
    code8 = _code(added_t, orig_t)
    code32 = lax.bitcast_convert_type(
        code8.reshape(B, W, HP // 4, 4), jnp.int32)
    codep = code32.reshape(B * W, HP // 4)
    parts = _sc_counts(codep, params)
    return jnp.sum(parts[:, 0]) * (1.0 / B)


# speedup vs baseline: 1.0813x; 1.0813x over previous
import functools

import jax
import jax.numpy as jnp
from jax import lax
from jax.experimental import pallas as pl
from jax.experimental.pallas import tpu as pltpu
from jax.experimental.pallas import tpu_sc as plsc

H, W, B, NB = 496, 432, 4, 50
INV = 1.25
RW = 40
NCC = H // 16
NBOX = 224
NWORK = 32
BPW = NBOX // NWORK
NROWS = RW * 4


def _boxparams_body(bx_ref, out_ref):
    cx = bx_ref[0, :]
    cy = bx_ref[1, :]
    cz = bx_ref[2, :]
    dx = bx_ref[3, :]
    dy = bx_ref[4, :]
    dz = bx_ref[5, :]
    rz = bx_ref[6, :]
    bidx = bx_ref[7, :]
    c = jnp.cos(rz)
    s = jnp.sin(rz)
    zok = jnp.logical_and(jnp.abs(cz) <= dz * 0.5, bidx >= 0.0)
    cxg = cx * INV
    cyg = cy * INV
    hxg = jnp.where(zok, dx * (0.5 * INV), -1.0)
    hyg = dy * (0.5 * INV)
    r0 = jnp.clip(jnp.floor(cyg) - 20.0, 0.0, float(W - RW))
    j0 = jnp.clip(jnp.floor((cxg - 18.0) / 16.0), 0.0, float((H - 64) // 16))
    bb = jnp.maximum(bidx, 0.0)
    base31j = (bb * float(W) + r0) * float(NCC) + j0
    out_ref[0, :] = cxg
    out_ref[1, :] = cyg
    out_ref[2, :] = c
    out_ref[3, :] = s
    out_ref[4, :] = hxg
    out_ref[5, :] = hyg
    out_ref[6, :] = r0
    out_ref[7, :] = j0 * 16.0
    out_ref[8, :] = base31j


def _boxparams(boxes):
    bx = jnp.transpose(boxes, (2, 0, 1)).reshape(7, B * NB)
    bxp = jnp.zeros((8, NBOX), jnp.float32)
    bxp = bxp.at[:7, : B * NB].set(bx)
    bidx = jnp.where(
        jnp.arange(NBOX) < B * NB, jnp.arange(NBOX) // NB, -1
    ).astype(jnp.float32)
    bxp = bxp.at[7, :].set(bidx)
    return pl.pallas_call(
        _boxparams_body,
        out_shape=jax.ShapeDtypeStruct((9, NBOX), jnp.float32),
    )(bxp)


TW1 = 216


def _code_body(added_ref, orig_ref, code_ref):
    pred = added_ref[0, 0] + added_ref[0, 1] + added_ref[0, 2] + added_ref[0, 3]
    og = orig_ref[0, 1] + orig_ref[0, 2] + orig_ref[0, 3] + orig_ref[0, 4]
    code_ref[0] = jnp.where(pred != 0.0, 1.0, 0.0) + jnp.where(og != 0.0, 2.0, 0.0)


def _code(added_t, orig_t):
    return pl.pallas_call(
        _code_body,
        grid=(B, W // TW1),
        in_specs=[
            pl.BlockSpec((1, 4, TW1, H), lambda b, w: (b, 0, w, 0)),
            pl.BlockSpec((1, 5, TW1, H), lambda b, w: (b, 0, w, 0)),
        ],
        out_specs=pl.BlockSpec((1, TW1, H), lambda b, w: (b, w, 0)),
        out_shape=jax.ShapeDtypeStruct((B, W, H), jnp.float32),
    )(added_t, orig_t)


def _splat_i(val):
    return jnp.full((16,), val, jnp.int32)


def _splat_f(val):
    return jnp.full((16,), val, jnp.float32)


def _sc_body(code_hbm, params_hbm, out_hbm, params_v, idx_a, idx_b, dst,
             outbuf, sem0, sem1):
    wid = lax.axis_index("s") * 2 + lax.axis_index("c")
    pltpu.sync_copy(params_hbm, params_v)
    iota = lax.broadcasted_iota(jnp.int32, (16,), 0)
    iota_f = iota.astype(jnp.float32)
    total_v = jnp.zeros((16,), jnp.float32)

    for k in range(BPW):
        box = wid + NWORK * k
        bsp = _splat_i(box)

        def _p(row):
            return plsc.load_gather(params_v, [_splat_i(row), bsp])

        cxg = _p(0)
        cyg = _p(1)
        c = _p(2)
        s = _p(3)
        hxg = _p(4)
        hyg = _p(5)
        r0f = _p(6)
        c0f = _p(7)
        base_i = _p(8).astype(jnp.int32)

        for t in range(10):
            e = iota + 16 * t
            iv = lax.shift_right_logical(e, 2)
            jj = lax.bitwise_and(e, 3)
            idxv = base_i + iv * NCC + jj
            if t < 5:
                idx_a[pl.ds(16 * t, 16)] = idxv
            else:
                idx_b[pl.ds(16 * (t - 5), 16)] = idxv

        cp1 = pltpu.async_copy(code_hbm.at[idx_a], dst.at[pl.ds(0, 80)], sem0)
        cp2 = pltpu.async_copy(code_hbm.at[idx_b], dst.at[pl.ds(80, 80)], sem1)
        cp1.wait()
        cp2.wait()

        u0 = c0f + iota_f - cxg
        v0 = r0f - cyg

        def _chunk_step(it, accs):
            acc_u, acc_i = accs
            dv0 = _splat_i(2 * it).astype(jnp.float32)
            for t in range(8):
                ch = it * 8 + t
                v = v0 + (dv0 + float(t // 4))
                u = u0 + float(16 * (t % 4))
                lx = u * c + v * s
                ly = v * c - u * s
                m = jnp.logical_and(jnp.abs(lx) <= hxg, jnp.abs(ly) <= hyg)
                codev = plsc.load_gather(dst, [_splat_i(ch), iota])
                acc_u = acc_u + jnp.where(
                    jnp.logical_and(m, codev != 0.0), 1.0, 0.0)
                acc_i = acc_i + jnp.where(
                    jnp.logical_and(m, codev == 3.0), 1.0, 0.0)
            return acc_u, acc_i

        acc_u, acc_i = lax.fori_loop(
            0, NROWS // 8, _chunk_step,
            (jnp.zeros((16,), jnp.float32), jnp.zeros((16,), jnp.float32)))

        us = jnp.sum(acc_u)
        isum = jnp.sum(acc_i)
        total_v = total_v + _splat_f(isum) / jnp.maximum(_splat_f(us), 1.0)

    outbuf[...] = total_v
    pltpu.sync_copy(outbuf, out_hbm.at[wid])


def _sc_counts(code16, params):
    mesh = plsc.VectorSubcoreMesh(core_axis_name="c", subcore_axis_name="s")
    f = functools.partial(
        pl.kernel,
        mesh=mesh,
        compiler_params=pltpu.CompilerParams(
            needs_layout_passes=False, use_tc_tiling_on_sc=False),
        out_type=jax.ShapeDtypeStruct((NWORK, 16), jnp.float32),
        scratch_types=[
            pltpu.VMEM((9, NBOX), jnp.float32),
            pltpu.VMEM((80,), jnp.int32),
            pltpu.VMEM((80,), jnp.int32),
            pltpu.VMEM((NROWS, 16), jnp.float32),
            pltpu.VMEM((16,), jnp.float32),
            pltpu.SemaphoreType.DMA,
            pltpu.SemaphoreType.DMA,
        ],
    )(_sc_body)
    return f(code16, params)


def kernel(added_points, original_points, boxes):
    params = _boxparams(boxes)
    added_t = jnp.swapaxes(added_points, 2, 3)
    orig_t = jnp.swapaxes(original_points, 2, 3)
    code = _code(added_t, orig_t)
    code16 = code.reshape(B * W * NCC, 16)
    parts = _sc_counts(code16, params)
    return jnp.sum(parts[:, 0]) * (1.0 / B)
